# baseline (device time: 13795 ns/iter reference)
import jax
import jax.numpy as jnp
from jax import lax
from jax.experimental import pallas as pl
from jax.experimental.pallas import tpu as pltpu

N_DEV = 4


def kernel(x, w_mat):
    m_per, k = x.shape
    n = w_mat.shape[1]
    n_per = n // N_DEV

    def body(x_hbm, w_hbm, out_hbm, x_vmem, w_vmem, out_vmem,
             send_buf, recv_buf, in_sems, out_sems, send_sems, recv_sems):
        my = lax.axis_index("i")

        x_dma = pltpu.make_async_copy(x_hbm, x_vmem, in_sems.at[0])
        x_dma.start()
        w_dmas = []
        for j in range(N_DEV):
            dma = pltpu.make_async_copy(
                w_hbm.at[:, pl.ds(j * n_per, n_per)],
                w_vmem.at[j],
                in_sems.at[1 + j],
            )
            dma.start()
            w_dmas.append(dma)

        with jax.named_scope("barrier"):
            barrier_sem = pltpu.get_barrier_semaphore()
            for d in range(1, N_DEV):
                pl.semaphore_signal(
                    barrier_sem, inc=1,
                    device_id=((my + d) % N_DEV,),
                    device_id_type=pl.DeviceIdType.MESH,
                )
            pl.semaphore_wait(barrier_sem, N_DEV - 1)

        with jax.named_scope("cast_x"):
            x_dma.wait()
            x_bf = x_vmem[:, :].astype(jnp.bfloat16)

        def send_desc(j, d):
            return pltpu.make_async_remote_copy(
                src_ref=send_buf.at[j],
                dst_ref=recv_buf.at[d],
                send_sem=send_sems.at[j],
                recv_sem=recv_sems.at[d],
                device_id=(j,),
                device_id_type=pl.DeviceIdType.MESH,
            )

        for j in range(N_DEV):
            with jax.named_scope(f"gemm#j={j}"):
                w_dmas[j].wait()
                w_bf = w_vmem[j].astype(jnp.bfloat16)
                y_j = jnp.dot(x_bf, w_bf, preferred_element_type=jnp.float32)
                send_buf[j] = y_j.astype(jnp.bfloat16)

            with jax.named_scope(f"send#j={j}"):
                @pl.when(j != my)
                def _():
                    send_desc(j, (j - my) % N_DEV).start()

                @pl.when(j == my)
                def _():
                    out_vmem[pl.ds(my * m_per, m_per), :] = y_j
                    pltpu.make_async_copy(
                        out_vmem.at[pl.ds(my * m_per, m_per), :],
                        out_hbm.at[pl.ds(my * m_per, m_per), :],
                        out_sems.at[0],
                    ).start()

        for d in range(1, N_DEV):
            with jax.named_scope(f"wait_recv#d={d}"):
                recv = pltpu.make_async_remote_copy(
                    src_ref=send_buf.at[0],
                    dst_ref=recv_buf.at[d],
                    send_sem=send_sems.at[0],
                    recv_sem=recv_sems.at[d],
                    device_id=(0,),
                    device_id_type=pl.DeviceIdType.MESH,
                )
                recv.wait_recv()
            with jax.named_scope(f"store#d={d}"):
                s = (my - d) % N_DEV
                out_vmem[pl.ds(s * m_per, m_per), :] = recv_buf[d].astype(jnp.float32)
                pltpu.make_async_copy(
                    out_vmem.at[pl.ds(s * m_per, m_per), :],
                    out_hbm.at[pl.ds(s * m_per, m_per), :],
                    out_sems.at[d],
                ).start()

        with jax.named_scope("drain"):
            for d in range(N_DEV):
                pltpu.make_async_copy(
                    out_vmem.at[pl.ds(0, m_per), :],
                    out_hbm.at[pl.ds(0, m_per), :],
                    out_sems.at[d],
                ).wait()
            for j in range(N_DEV):
                @pl.when(j != my)
                def _():
                    send_desc(j, (j - my) % N_DEV).wait_send()

    out_shape = jax.ShapeDtypeStruct((N_DEV * m_per, n_per), jnp.float32)
    return pl.pallas_call(
        body,
        out_shape=out_shape,
        in_specs=[
            pl.BlockSpec(memory_space=pl.ANY),
            pl.BlockSpec(memory_space=pl.ANY),
        ],
        out_specs=pl.BlockSpec(memory_space=pl.ANY),
        scratch_shapes=[
            pltpu.VMEM((m_per, k), jnp.float32),
            pltpu.VMEM((N_DEV, k, n_per), jnp.float32),
            pltpu.VMEM((N_DEV * m_per, n_per), jnp.float32),
            pltpu.VMEM((N_DEV, m_per, n_per), jnp.bfloat16),
            pltpu.VMEM((N_DEV, m_per, n_per), jnp.bfloat16),
            pltpu.SemaphoreType.DMA((1 + N_DEV,)),
            pltpu.SemaphoreType.DMA((N_DEV,)),
            pltpu.SemaphoreType.DMA((N_DEV,)),
            pltpu.SemaphoreType.DMA((N_DEV,)),
        ],
        compiler_params=pltpu.CompilerParams(collective_id=0),
    )(x, w_mat)


# device time: 12405 ns/iter; 1.1121x vs baseline; 1.1121x over previous
import jax
import jax.numpy as jnp
from jax import lax
from jax.experimental import pallas as pl
from jax.experimental.pallas import tpu as pltpu

N_DEV = 4


def kernel(x, w_mat):
    m_per, k = x.shape
    n = w_mat.shape[1]
    n_per = n // N_DEV

    def body(x_hbm, w_hbm, out_hbm, x_vmem, w_vmem, out_vmem,
             send_buf, recv_buf, in_sems, out_sems, send_sems, recv_sems):
        my = lax.axis_index("i")

        x_dma = pltpu.make_async_copy(x_hbm, x_vmem, in_sems.at[0])
        x_dma.start()
        w_dmas = []
        for j in range(N_DEV):
            dma = pltpu.make_async_copy(
                w_hbm.at[:, pl.ds(j * n_per, n_per)],
                w_vmem.at[j],
                in_sems.at[1 + j],
            )
            dma.start()
            w_dmas.append(dma)

        with jax.named_scope("barrier"):
            barrier_sem = pltpu.get_barrier_semaphore()
            for d in range(1, N_DEV):
                pl.semaphore_signal(
                    barrier_sem, inc=1,
                    device_id=((my + d) % N_DEV,),
                    device_id_type=pl.DeviceIdType.MESH,
                )
            pl.semaphore_wait(barrier_sem, N_DEV - 1)

        with jax.named_scope("cast_x"):
            x_dma.wait()
            x_bf = x_vmem[:, :].astype(jnp.bfloat16)

        def send_desc(j, d):
            return pltpu.make_async_remote_copy(
                src_ref=send_buf.at[j],
                dst_ref=recv_buf.at[d],
                send_sem=send_sems.at[j],
                recv_sem=recv_sems.at[d],
                device_id=(j,),
                device_id_type=pl.DeviceIdType.MESH,
            )

        for j in range(N_DEV):
            with jax.named_scope(f"gemm#j={j}"):
                w_dmas[j].wait()
                w_bf = w_vmem[j].astype(jnp.bfloat16)
                y_j = jnp.dot(x_bf, w_bf, preferred_element_type=jnp.float32)
                send_buf[j] = y_j.astype(jnp.bfloat16)

            with jax.named_scope(f"send#j={j}"):
                @pl.when(j != my)
                def _():
                    send_desc(j, (j - my) % N_DEV).start()

                @pl.when(j == my)
                def _():
                    out_vmem[pl.ds(my * m_per, m_per), :] = y_j
                    pltpu.make_async_copy(
                        out_vmem.at[pl.ds(my * m_per, m_per), :],
                        out_hbm.at[pl.ds(my * m_per, m_per), :],
                        out_sems.at[0],
                    ).start()

        for d in range(1, N_DEV):
            with jax.named_scope(f"wait_recv#d={d}"):
                recv = pltpu.make_async_remote_copy(
                    src_ref=send_buf.at[0],
                    dst_ref=recv_buf.at[d],
                    send_sem=send_sems.at[0],
                    recv_sem=recv_sems.at[d],
                    device_id=(0,),
                    device_id_type=pl.DeviceIdType.MESH,
                )
                recv.wait_recv()
            with jax.named_scope(f"store#d={d}"):
                s = (my - d) % N_DEV
                out_vmem[pl.ds(s * m_per, m_per), :] = recv_buf[d].astype(jnp.float32)
                pltpu.make_async_copy(
                    out_vmem.at[pl.ds(s * m_per, m_per), :],
                    out_hbm.at[pl.ds(s * m_per, m_per), :],
                    out_sems.at[d],
                ).start()

        with jax.named_scope("drain"):
            for d in range(N_DEV):
                pltpu.make_async_copy(
                    out_vmem.at[pl.ds(0, m_per), :],
                    out_hbm.at[pl.ds(0, m_per), :],
                    out_sems.at[d],
                ).wait()
            for j in range(N_DEV):
                @pl.when(j != my)
                def _():
                    send_desc(j, (j - my) % N_DEV).wait_send()

    out_shape = jax.ShapeDtypeStruct((N_DEV * m_per, n_per), jnp.float32)
    return pl.pallas_call(
        body,
        out_shape=out_shape,
        in_specs=[
            pl.BlockSpec(memory_space=pltpu.MemorySpace.HBM),
            pl.BlockSpec(memory_space=pltpu.MemorySpace.HBM),
        ],
        out_specs=pl.BlockSpec(memory_space=pltpu.MemorySpace.HBM),
        scratch_shapes=[
            pltpu.VMEM((m_per, k), jnp.float32),
            pltpu.VMEM((N_DEV, k, n_per), jnp.float32),
            pltpu.VMEM((N_DEV * m_per, n_per), jnp.float32),
            pltpu.VMEM((N_DEV, m_per, n_per), jnp.bfloat16),
            pltpu.VMEM((N_DEV, m_per, n_per), jnp.bfloat16),
            pltpu.SemaphoreType.DMA((1 + N_DEV,)),
            pltpu.SemaphoreType.DMA((N_DEV,)),
            pltpu.SemaphoreType.DMA((N_DEV,)),
            pltpu.SemaphoreType.DMA((N_DEV,)),
        ],
        compiler_params=pltpu.CompilerParams(collective_id=0),
    )(x, w_mat)


# device time: 9754 ns/iter; 1.4143x vs baseline; 1.2718x over previous
import jax
import jax.numpy as jnp
from jax import lax
from jax.experimental import pallas as pl
from jax.experimental.pallas import tpu as pltpu

N_DEV = 4


def kernel(x, w_mat):
    m_per, k = x.shape
    n = w_mat.shape[1]
    n_per = n // N_DEV

    def body(x_hbm, w_hbm, out_ref, x_vmem, w_vmem,
             send_buf, recv_buf, in_sems, send_sems, recv_sems):
        my = lax.axis_index("i")

        x_dma = pltpu.make_async_copy(x_hbm, x_vmem, in_sems.at[0])
        x_dma.start()
        w_dmas = []
        for j in range(N_DEV):
            dma = pltpu.make_async_copy(
                w_hbm.at[:, pl.ds(j * n_per, n_per)],
                w_vmem.at[j],
                in_sems.at[1 + j],
            )
            dma.start()
            w_dmas.append(dma)

        with jax.named_scope("barrier"):
            barrier_sem = pltpu.get_barrier_semaphore()
            for d in range(1, N_DEV):
                pl.semaphore_signal(
                    barrier_sem, inc=1,
                    device_id=((my + d) % N_DEV,),
                    device_id_type=pl.DeviceIdType.MESH,
                )
            pl.semaphore_wait(barrier_sem, N_DEV - 1)

        with jax.named_scope("cast_x"):
            x_dma.wait()
            x_bf = x_vmem[:, :].astype(jnp.bfloat16)

        def send_desc(j, d):
            return pltpu.make_async_remote_copy(
                src_ref=send_buf.at[j],
                dst_ref=recv_buf.at[d],
                send_sem=send_sems.at[j],
                recv_sem=recv_sems.at[d],
                device_id=(j,),
                device_id_type=pl.DeviceIdType.MESH,
            )

        for j in range(N_DEV):
            with jax.named_scope(f"gemm#j={j}"):
                w_dmas[j].wait()
                w_bf = w_vmem[j].astype(jnp.bfloat16)
                y_j = jnp.dot(x_bf, w_bf, preferred_element_type=jnp.float32)
                send_buf[j] = y_j.astype(jnp.bfloat16)

            with jax.named_scope(f"send#j={j}"):
                @pl.when(j != my)
                def _():
                    send_desc(j, (j - my) % N_DEV).start()

                @pl.when(j == my)
                def _():
                    out_ref[pl.ds(my * m_per, m_per), :] = y_j

        for d in range(1, N_DEV):
            with jax.named_scope(f"wait_recv#d={d}"):
                recv = pltpu.make_async_remote_copy(
                    src_ref=send_buf.at[0],
                    dst_ref=recv_buf.at[d],
                    send_sem=send_sems.at[0],
                    recv_sem=recv_sems.at[d],
                    device_id=(0,),
                    device_id_type=pl.DeviceIdType.MESH,
                )
                recv.wait_recv()
            with jax.named_scope(f"store#d={d}"):
                s = (my - d) % N_DEV
                out_ref[pl.ds(s * m_per, m_per), :] = recv_buf[d].astype(jnp.float32)

        with jax.named_scope("drain"):
            for j in range(N_DEV):
                @pl.when(j != my)
                def _():
                    send_desc(j, (j - my) % N_DEV).wait_send()

    out_shape = jax.ShapeDtypeStruct((N_DEV * m_per, n_per), jnp.float32)
    x = pltpu.with_memory_space_constraint(x, pltpu.MemorySpace.HBM)
    w_mat = pltpu.with_memory_space_constraint(w_mat, pltpu.MemorySpace.HBM)
    return pl.pallas_call(
        body,
        out_shape=out_shape,
        in_specs=[
            pl.BlockSpec(memory_space=pltpu.MemorySpace.HBM),
            pl.BlockSpec(memory_space=pltpu.MemorySpace.HBM),
        ],
        out_specs=pl.BlockSpec(memory_space=pltpu.VMEM),
        scratch_shapes=[
            pltpu.VMEM((m_per, k), jnp.float32),
            pltpu.VMEM((N_DEV, k, n_per), jnp.float32),
            pltpu.VMEM((N_DEV, m_per, n_per), jnp.bfloat16),
            pltpu.VMEM((N_DEV, m_per, n_per), jnp.bfloat16),
            pltpu.SemaphoreType.DMA((1 + N_DEV,)),
            pltpu.SemaphoreType.DMA((N_DEV,)),
            pltpu.SemaphoreType.DMA((N_DEV,)),
        ],
        compiler_params=pltpu.CompilerParams(collective_id=0),
    )(x, w_mat)
